# baseline (speedup 1.0000x reference)
import functools

import jax
import jax.numpy as jnp
from jax import lax
from jax.experimental import pallas as pl
from jax.experimental.pallas import tpu as pltpu
from jax.experimental.pallas import tpu_sc as plsc

EMBED_D = 64
NUM_CORES = 2
NUM_SUBCORES = 16
NUM_WORKERS = NUM_CORES * NUM_SUBCORES
BBLK = 128


def _transpose_block(g_ref, gt_ref):
  for j in range(BBLK // 16):
    rows = lax.iota(jnp.int32, 16) + (j * 16)
    for d in range(EMBED_D):
      col = jnp.full((16,), d, jnp.int32)
      v = plsc.load_gather(g_ref, [rows, col])
      gt_ref[d // 8, d % 8, pl.ds(j * 16, 16)] = v


def _make_lookup(batch: int, seq: int):
  assert batch % NUM_WORKERS == 0 and batch // NUM_WORKERS == BBLK
  n_bt = batch // BBLK
  assert seq % 2 == 0

  mesh = plsc.VectorSubcoreMesh(
      core_axis_name="c", subcore_axis_name="s", num_cores=NUM_CORES)

  @functools.partial(
      pl.kernel,
      out_type=jax.ShapeDtypeStruct((seq, 8, n_bt, 8, BBLK), jnp.float32),
      mesh=mesh,
      compiler_params=pltpu.CompilerParams(
          use_tc_tiling_on_sc=False, needs_layout_passes=False),
      scratch_types=[
          pltpu.VMEM((seq, BBLK), jnp.int32),
          pltpu.VMEM((BBLK, EMBED_D), jnp.float32),
          pltpu.VMEM((BBLK, EMBED_D), jnp.float32),
          pltpu.VMEM((8, 8, BBLK), jnp.float32),
          pltpu.VMEM((8, 8, BBLK), jnp.float32),
          pltpu.SemaphoreType.DMA,
          pltpu.SemaphoreType.DMA,
          pltpu.SemaphoreType.DMA,
      ],
  )
  def lookup(table_hbm, idst_hbm, out_hbm, idx_v, g0, g1, gt0, gt1, gsem,
             ssem0, ssem1):
    wid = lax.axis_index("s") * NUM_CORES + lax.axis_index("c")
    b0 = wid * BBLK

    pltpu.sync_copy(idst_hbm.at[:, pl.ds(b0, BBLK)], idx_v)

    def gather(s, g):
      return pltpu.async_copy(table_hbm.at[idx_v.at[s]], g, gsem)

    def wait_gather(s, g):
      pltpu.make_async_copy(table_hbm.at[idx_v.at[s]], g, gsem).wait()

    def store(s, gt, ssem):
      return pltpu.async_copy(gt, out_hbm.at[s].at[:, wid], ssem)

    gather(0, g0)

    def body(h, carry):
      for b, (g, gt, ssem) in enumerate(
          ((g0, gt0, ssem0), (g1, gt1, ssem1))):
        s = h * 2 + b
        wait_gather(s, g)

        nxt = s + 1
        if b == 0:
          gather(nxt, g1)
        else:
          @pl.when(h < (seq // 2 - 1))
          def _fire_next():
            gather(nxt, g0)

        @pl.when(h > 0)
        def _wait_prev_store():
          pltpu.make_async_copy(gt, out_hbm.at[s - 2].at[:, wid], ssem).wait()

        _transpose_block(g, gt)
        store(s, gt, ssem)
      return carry

    lax.fori_loop(0, seq // 2, body, 0)

    pltpu.make_async_copy(gt0, out_hbm.at[seq - 2].at[:, wid], ssem0).wait()
    pltpu.make_async_copy(gt1, out_hbm.at[seq - 1].at[:, wid], ssem1).wait()

  return lookup


def kernel(ids, W):
  batch, seq = ids.shape
  idst = ids.T.astype(jnp.int32)
  o5 = _make_lookup(batch, seq)(W, idst)
  op = jnp.transpose(o5, (0, 1, 3, 2, 4)).reshape(seq, EMBED_D, batch)
  return jnp.transpose(op, (2, 0, 1))


# speedup vs baseline: 1.7598x; 1.7598x over previous
import functools

import jax
import jax.numpy as jnp
from jax import lax
from jax.experimental import pallas as pl
from jax.experimental.pallas import tpu as pltpu
from jax.experimental.pallas import tpu_sc as plsc

EMBED_D = 64
NUM_CORES = 2
NUM_SUBCORES = 16
NUM_WORKERS = NUM_CORES * NUM_SUBCORES
CHUNK = 128
K = 5
GROUP = CHUNK * K


def _make_lookup(total_rows: int):
  rows_per_w = total_rows // NUM_WORKERS
  idx_rows_per_w = rows_per_w // CHUNK
  n_groups = idx_rows_per_w // K
  assert rows_per_w % (CHUNK * K) == 0 and n_groups >= 2

  mesh = plsc.VectorSubcoreMesh(
      core_axis_name="c", subcore_axis_name="s", num_cores=NUM_CORES)

  @functools.partial(
      pl.kernel,
      out_type=jax.ShapeDtypeStruct((total_rows, EMBED_D), jnp.float32),
      mesh=mesh,
      compiler_params=pltpu.CompilerParams(use_tc_tiling_on_sc=False),
      scratch_types=[
          pltpu.VMEM((idx_rows_per_w, CHUNK), jnp.int32),
          pltpu.VMEM((GROUP, EMBED_D), jnp.float32),
          pltpu.VMEM((GROUP, EMBED_D), jnp.float32),
          pltpu.SemaphoreType.DMA,
          pltpu.SemaphoreType.DMA,
          pltpu.SemaphoreType.DMA,
          pltpu.SemaphoreType.DMA,
      ],
  )
  def lookup(table_hbm, idx_hbm, out_hbm, idx_v, rows0, rows1, gsem0, gsem1,
             ssem0, ssem1):
    wid = lax.axis_index("s") * NUM_CORES + lax.axis_index("c")
    idx_base = wid * idx_rows_per_w
    out_base = wid * rows_per_w

    pltpu.sync_copy(idx_hbm.at[pl.ds(idx_base, idx_rows_per_w)], idx_v)

    bufs = ((rows0, gsem0, ssem0), (rows1, gsem1, ssem1))

    def fire_gathers(g, rows_v, gsem):
      for j in range(K):
        pltpu.async_copy(
            table_hbm.at[idx_v.at[g * K + j]],
            rows_v.at[pl.ds(j * CHUNK, CHUNK)], gsem)

    def wait_gathers(g, rows_v, gsem):
      for j in range(K):
        pltpu.make_async_copy(
            table_hbm.at[idx_v.at[g * K + j]],
            rows_v.at[pl.ds(j * CHUNK, CHUNK)], gsem).wait()

    def out_slice(g):
      return out_hbm.at[pl.ds(out_base + g * GROUP, GROUP)]

    fire_gathers(0, rows0, gsem0)
    for g in range(n_groups):
      rows_v, gsem, ssem = bufs[g % 2]
      if g + 1 < n_groups:
        rows_n, gsem_n, ssem_n = bufs[(g + 1) % 2]
        if g >= 1:
          pltpu.make_async_copy(rows_n, out_slice(g - 1), ssem_n).wait()
        fire_gathers(g + 1, rows_n, gsem_n)
      wait_gathers(g, rows_v, gsem)
      pltpu.async_copy(rows_v, out_slice(g), ssem)

    r2, _, s2 = bufs[(n_groups - 2) % 2]
    r1, _, s1 = bufs[(n_groups - 1) % 2]
    pltpu.make_async_copy(r2, out_slice(n_groups - 2), s2).wait()
    pltpu.make_async_copy(r1, out_slice(n_groups - 1), s1).wait()

  return lookup


def kernel(ids, W):
  flat_ids = ids.reshape(-1).astype(jnp.int32)
  total_rows = flat_ids.shape[0]
  idx2d = flat_ids.reshape(total_rows // CHUNK, CHUNK)
  out = _make_lookup(total_rows)(W, idx2d)
  return out.reshape(ids.shape + (EMBED_D,))
